# baseline (speedup 1.0000x reference)
import functools

import jax
import jax.numpy as jnp
from jax import lax
from jax.experimental import pallas as pl
from jax.experimental.pallas import tpu as pltpu
from jax.experimental.pallas import tpu_sc as plsc

VOCAB_N = 100000
EMB_D = 64
HID_D = 128
NIDX = 16384

NC = 2
NS = 16
NW = NC * NS

LANES = 100096
VALID = VOCAB_N
RUN_L = 256
N_RUNS = LANES // RUN_L
RUNS_PER_S = 25
HALF_L = RUN_L // 2
HCHUNKS = HALF_L // 16
DIMS_PER_C = EMB_D // NC
IDX_PER_S = NIDX // NS
PAD_RUN = (VALID // RUN_L)
PAD_CHUNK = (VALID - PAD_RUN * RUN_L - HALF_L) // 16


def _sc_ctx(idx_hbm, emt_hbm, out_hbm,
            idx_v, ones_v, zeros_v, cnt0, cnt1, buf0, buf1, acc_v, counts, sem):
    c = lax.axis_index("c")
    s = lax.axis_index("s")
    wid = c * NS + s
    row0 = c * DIMS_PER_C

    for k in range(8):
        ones_v[pl.ds(k * 16, 16)] = jnp.ones((16,), jnp.float32)
    zslice = 6144
    for k in range(zslice // 16):
        zeros_v[pl.ds(k * 16, 16)] = jnp.zeros((16,), jnp.float32)
    pltpu.sync_copy(zeros_v, counts.at[pl.ds(s * zslice, zslice)])

    @pl.when(s == 0)
    def _():
        rem = LANES - NS * zslice
        pltpu.sync_copy(zeros_v.at[pl.ds(0, rem)],
                        counts.at[pl.ds(NS * zslice, rem)])

    plsc.subcore_barrier()

    pltpu.sync_copy(idx_hbm.at[s], idx_v)
    for j in range(IDX_PER_S // 128):
        pltpu.sync_copy(ones_v, counts.at[idx_v.at[j]], add=True)
    plsc.subcore_barrier()

    pltpu.async_copy(
        emt_hbm.at[pl.ds(row0, DIMS_PER_C), pl.ds(s * RUN_L, HALF_L)],
        buf0, sem)

    def mac(buf, cnt, accs):
        out = list(accs)
        for j in range(HCHUNKS):
            cv = cnt[pl.ds(j * 16, 16)]
            for d in range(DIMS_PER_C):
                out[d] = out[d] + buf[d, pl.ds(j * 16, 16)] * cv
        return out

    def k_body(k, carry):
        run_id = s + NS * k
        lane0 = run_id * RUN_L
        valid = run_id < N_RUNS

        @pl.when(valid)
        def _():
            pltpu.make_async_copy(
                emt_hbm.at[pl.ds(row0, DIMS_PER_C), pl.ds(lane0, HALF_L)],
                buf0, sem).wait()
            pltpu.async_copy(
                emt_hbm.at[pl.ds(row0, DIMS_PER_C),
                           pl.ds(lane0 + HALF_L, HALF_L)],
                buf1, sem)
            pltpu.sync_copy(counts.at[pl.ds(lane0, HALF_L)], cnt0)

        @pl.when(jnp.logical_not(valid))
        def _():
            for j in range(HCHUNKS):
                cnt0[pl.ds(j * 16, 16)] = jnp.zeros((16,), jnp.float32)

        accs = mac(buf0, cnt0, carry)

        @pl.when(valid)
        def _():
            pltpu.make_async_copy(
                emt_hbm.at[pl.ds(row0, DIMS_PER_C),
                           pl.ds(lane0 + HALF_L, HALF_L)],
                buf1, sem).wait()
            pltpu.sync_copy(counts.at[pl.ds(lane0 + HALF_L, HALF_L)], cnt1)

        next_valid = jnp.logical_and(k + 1 < RUNS_PER_S,
                                     s + NS * (k + 1) < N_RUNS)

        @pl.when(next_valid)
        def _():
            pltpu.async_copy(
                emt_hbm.at[pl.ds(row0, DIMS_PER_C),
                           pl.ds((s + NS * (k + 1)) * RUN_L, HALF_L)],
                buf0, sem)

        @pl.when(jnp.logical_not(valid))
        def _():
            for j in range(HCHUNKS):
                cnt1[pl.ds(j * 16, 16)] = jnp.zeros((16,), jnp.float32)

        @pl.when(run_id == PAD_RUN)
        def _():
            for d in range(DIMS_PER_C):
                for j in range(PAD_CHUNK, HCHUNKS):
                    buf1[d, pl.ds(j * 16, 16)] = jnp.zeros((16,), jnp.float32)

        accs = mac(buf1, cnt1, accs)
        return tuple(accs)

    zero = jnp.zeros((16,), jnp.float32)
    accs = list(lax.fori_loop(0, RUNS_PER_S, k_body,
                              tuple(zero for _ in range(DIMS_PER_C))))

    for d in range(DIMS_PER_C):
        acc_v[0, d] = accs[d]
    pltpu.sync_copy(acc_v, out_hbm.at[pl.ds(wid, 1)])


_sc_mesh = plsc.VectorSubcoreMesh(core_axis_name="c", subcore_axis_name="s")

_ctx_sum = functools.partial(
    pl.kernel,
    out_type=jax.ShapeDtypeStruct((NW, DIMS_PER_C, 16), jnp.float32),
    mesh=_sc_mesh,
    scratch_types=[
        pltpu.VMEM((NS // 2, 128), jnp.int32),
        pltpu.VMEM((128,), jnp.float32),
        pltpu.VMEM((6144,), jnp.float32),
        pltpu.VMEM((HALF_L,), jnp.float32),
        pltpu.VMEM((HALF_L,), jnp.float32),
        pltpu.VMEM((DIMS_PER_C, HALF_L), jnp.float32),
        pltpu.VMEM((DIMS_PER_C, HALF_L), jnp.float32),
        pltpu.VMEM((1, DIMS_PER_C, 16), jnp.float32),
        pltpu.VMEM_SHARED((LANES,), jnp.float32),
        pltpu.SemaphoreType.DMA,
    ],
    compiler_params=pltpu.CompilerParams(use_tc_tiling_on_sc=True),
)(_sc_ctx)


BLOCK_V = 10000
NBLK = VOCAB_N // BLOCK_V


def _tc_logits_body(part_ref, w1_ref, b1_ref, w2_ref, b2_ref,
                    logit_ref, lse_ref, h_ref, ms_ref):
    i = pl.program_id(0)

    @pl.when(i == 0)
    def _():
        t = jnp.sum(part_ref[...], axis=2)
        low = jnp.sum(t[:NS], axis=0)
        high = jnp.sum(t[NS:], axis=0)
        ctx = jnp.concatenate([low, high]).reshape(1, EMB_D)
        h = lax.dot_general(ctx, w1_ref[...], (((1,), (1,)), ((), ())),
                            preferred_element_type=jnp.float32)
        h_ref[...] = jnp.maximum(h + b1_ref[...], 0.0)
        ms_ref[0] = -jnp.inf
        ms_ref[1] = 0.0

    logits = lax.dot_general(h_ref[...], w2_ref[...], (((1,), (1,)), ((), ())),
                             preferred_element_type=jnp.float32)
    logits = logits + b2_ref[...].reshape(1, BLOCK_V)
    logit_ref[...] = logits.reshape(1, 1, BLOCK_V)

    m_old = ms_ref[0]
    s_old = ms_ref[1]
    m_new = jnp.maximum(m_old, jnp.max(logits))
    s_new = s_old * jnp.exp(m_old - m_new) + jnp.sum(jnp.exp(logits - m_new))
    ms_ref[0] = m_new
    ms_ref[1] = s_new

    @pl.when(i == NBLK - 1)
    def _():
        lse_ref[0, 0] = m_new + jnp.log(s_new)


_tc_logits = pl.pallas_call(
    _tc_logits_body,
    grid=(NBLK,),
    in_specs=[
        pl.BlockSpec((NW, DIMS_PER_C, 16), lambda i: (0, 0, 0)),
        pl.BlockSpec((HID_D, EMB_D), lambda i: (0, 0)),
        pl.BlockSpec((1, HID_D), lambda i: (0, 0)),
        pl.BlockSpec((BLOCK_V, HID_D), lambda i: (i, 0)),
        pl.BlockSpec((1, 1, BLOCK_V), lambda i: (i, 0, 0)),
    ],
    out_specs=[
        pl.BlockSpec((1, 1, BLOCK_V), lambda i: (i, 0, 0)),
        pl.BlockSpec(memory_space=pltpu.SMEM),
    ],
    out_shape=[
        jax.ShapeDtypeStruct((NBLK, 1, BLOCK_V), jnp.float32),
        jax.ShapeDtypeStruct((1, 1), jnp.float32),
    ],
    scratch_shapes=[
        pltpu.VMEM((1, HID_D), jnp.float32),
        pltpu.SMEM((2,), jnp.float32),
    ],
)


def _tc_sub_body(logit_ref, lse_ref, out_ref):
    out_ref[...] = logit_ref[...] - lse_ref[0, 0]


_tc_sub = pl.pallas_call(
    _tc_sub_body,
    in_specs=[
        pl.BlockSpec(memory_space=pltpu.VMEM),
        pl.BlockSpec(memory_space=pltpu.SMEM),
    ],
    out_specs=pl.BlockSpec(memory_space=pltpu.VMEM),
    out_shape=jax.ShapeDtypeStruct((NBLK, 1, BLOCK_V), jnp.float32),
)


def kernel(inputs, emb, W1, b1, W2, b2):
    idx = inputs.astype(jnp.int32).reshape(NS, NIDX // NS // 128, 128)
    partials = _ctx_sum(idx, emb.T)
    logits, lse = _tc_logits(partials, W1, b1.reshape(1, HID_D),
                             W2, b2.reshape(NBLK, 1, BLOCK_V))
    return _tc_sub(logits, lse).reshape(1, VOCAB_N)
